# baseline (speedup 1.0000x reference)
import functools

import jax
import jax.numpy as jnp
from jax import lax
from jax.experimental import pallas as pl
from jax.experimental.pallas import tpu as pltpu
from jax.experimental.pallas import tpu_sc as plsc

N_CAT = 26
CARD1 = 100001
CAT_DIM = 36
N_NUM = 13
NUM_DIM = 8
CAT_OUT = N_CAT * CAT_DIM
NUM_OUT = N_NUM * NUM_DIM
IDXW = 128
SUP = 2 * CAT_DIM


@functools.lru_cache(maxsize=None)
def _make_gather(B: int):
    info = plsc.get_sparse_core_info()
    NC, NS = info.num_cores, info.num_subcores
    NW = NC * NS
    TOTAL = B * N_CAT
    NROWS = TOTAL // IDXW
    per_w = NROWS // NW
    G = 8
    CHUNKS = per_w // G

    mesh = plsc.VectorSubcoreMesh(core_axis_name="c", subcore_axis_name="s")

    @functools.partial(
        pl.kernel,
        mesh=mesh,
        out_type=jax.ShapeDtypeStruct((TOTAL, SUP), jnp.float32),
        scratch_types=(
            [pltpu.VMEM((IDXW,), jnp.int32) for _ in range(G)]
            + [pltpu.VMEM((IDXW, SUP), jnp.float32) for _ in range(G)]
            + [pltpu.SemaphoreType.DMA]
        ),
        compiler_params=pltpu.CompilerParams(use_tc_tiling_on_sc=False),
    )
    def gather_k(tbl_hbm, idx_hbm, out_hbm, *scratch):
        idx_vs = scratch[:G]
        rows_vs = scratch[G:2 * G]
        sem = scratch[2 * G]
        wid = lax.axis_index("s") * NC + lax.axis_index("c")

        def body(g, carry):
            r0 = pl.multiple_of((wid * per_w + g * G) * IDXW, IDXW)
            for j in range(G):
                pltpu.sync_copy(
                    idx_hbm.at[pl.ds(pl.multiple_of(r0 + j * IDXW, IDXW), IDXW)],
                    idx_vs[j])
            copies = [
                pltpu.async_copy(tbl_hbm.at[idx_vs[j]], rows_vs[j], sem)
                for j in range(G)
            ]
            for c in copies:
                c.wait()
            for j in range(G):
                pltpu.sync_copy(
                    rows_vs[j],
                    out_hbm.at[pl.ds(pl.multiple_of(r0 + j * IDXW, IDXW), IDXW)])
            return carry

        lax.fori_loop(0, CHUNKS, body, 0)

    return gather_k


def _combine(sup_flat, par, num_values, smat, bias, B):
    BB = 1024
    D = CAT_OUT + NUM_OUT

    def body(sup_ref, par_ref, nv_ref, s_ref, b_ref, out_ref):
        x = sup_ref[...]
        p = par_ref[...]
        parts = []
        for g in range(N_CAT):
            lo = x[:, g * SUP:g * SUP + CAT_DIM]
            hi = x[:, g * SUP + CAT_DIM:(g + 1) * SUP]
            pg = p[:, g:g + 1]
            parts.append(jnp.where(pg > 0.5, hi, lo))
        num = jax.nn.relu(
            jnp.dot(nv_ref[...], s_ref[...],
                    preferred_element_type=jnp.float32,
                    precision=jax.lax.Precision.HIGHEST) + b_ref[...])
        out_ref[...] = jnp.concatenate(parts + [num], axis=1)

    return pl.pallas_call(
        body,
        grid=(B // BB,),
        in_specs=[
            pl.BlockSpec((BB, N_CAT * SUP), lambda i: (i, 0)),
            pl.BlockSpec((BB, N_CAT), lambda i: (i, 0)),
            pl.BlockSpec((BB, N_NUM), lambda i: (i, 0)),
            pl.BlockSpec((N_NUM, NUM_OUT), lambda i: (0, 0)),
            pl.BlockSpec((1, NUM_OUT), lambda i: (0, 0)),
        ],
        out_specs=pl.BlockSpec((BB, D), lambda i: (i, 0)),
        out_shape=jax.ShapeDtypeStruct((B, D), jnp.float32),
    )(sup_flat, par, num_values, smat, bias)


def kernel(cat_indices, num_values, cat_tables, num_w, num_b):
    B = cat_indices.shape[0]
    tbl = cat_tables.reshape(N_CAT * CARD1 // 2, SUP)
    offs = (jnp.arange(N_CAT, dtype=jnp.int32) * CARD1)[None, :]
    idx = cat_indices + offs
    sidx = (idx >> 1).reshape(B * N_CAT)
    par = (idx & 1).astype(jnp.float32)
    sup = _make_gather(B)(tbl, sidx)
    sup_flat = sup.reshape(B, N_CAT * SUP)
    j = jnp.arange(N_NUM)
    smat = jnp.zeros((N_NUM, NUM_OUT), jnp.float32).at[
        j[:, None], j[:, None] * NUM_DIM + jnp.arange(NUM_DIM)[None, :]
    ].set(num_w)
    bias = num_b.reshape(1, NUM_OUT)
    return _combine(sup_flat, par, num_values, smat, bias, B)


# speedup vs baseline: 3.7863x; 3.7863x over previous
import functools

import jax
import jax.numpy as jnp
from jax import lax
from jax.experimental import pallas as pl
from jax.experimental.pallas import tpu as pltpu
from jax.experimental.pallas import tpu_sc as plsc

N_CAT = 26
CARD1 = 100001
CAT_DIM = 36
N_NUM = 13
NUM_DIM = 8
CAT_OUT = N_CAT * CAT_DIM
NUM_OUT = N_NUM * NUM_DIM
IDXW = 128
LANES = 128

PACK_SUP = 1024
PACK_STEPS = 49
SUP_PER_SLAB = PACK_SUP * PACK_STEPS


def _pack_table(tbl_t):
    def body(lo_ref, hi_ref, o_ref):
        lo = lo_ref[0]
        hi = hi_ref[0]
        o_ref[0] = jnp.concatenate(
            [lo.T, hi.T, jnp.zeros((PACK_SUP, LANES - 2 * CAT_DIM),
                                   jnp.float32)], axis=1)

    return pl.pallas_call(
        body,
        grid=(N_CAT, PACK_STEPS),
        in_specs=[
            pl.BlockSpec((1, CAT_DIM, PACK_SUP), lambda i, j: (i, 0, j)),
            pl.BlockSpec((1, CAT_DIM, PACK_SUP),
                         lambda i, j: (i, 0, j + PACK_STEPS)),
        ],
        out_specs=pl.BlockSpec((1, PACK_SUP, LANES), lambda i, j: (i, j, 0)),
        out_shape=jax.ShapeDtypeStruct((N_CAT, SUP_PER_SLAB, LANES),
                                       jnp.float32),
    )(tbl_t, tbl_t)


@functools.lru_cache(maxsize=None)
def _make_gather(B: int):
    info = plsc.get_sparse_core_info()
    NC, NS = info.num_cores, info.num_subcores
    NW = NC * NS
    TOTAL = B * N_CAT
    NROWS = TOTAL // IDXW
    per_w = NROWS // NW
    G = 4
    CHUNKS = per_w // G

    mesh = plsc.VectorSubcoreMesh(core_axis_name="c", subcore_axis_name="s")

    @functools.partial(
        pl.kernel,
        mesh=mesh,
        out_type=jax.ShapeDtypeStruct((TOTAL, LANES), jnp.float32),
        scratch_types=(
            [pltpu.VMEM((IDXW,), jnp.int32) for _ in range(G)]
            + [pltpu.VMEM((IDXW, LANES), jnp.float32) for _ in range(G)]
            + [pltpu.SemaphoreType.DMA]
        ),
    )
    def gather_k(tbl_hbm, idx_hbm, out_hbm, *scratch):
        idx_vs = scratch[:G]
        rows_vs = scratch[G:2 * G]
        sem = scratch[2 * G]
        wid = lax.axis_index("s") * NC + lax.axis_index("c")

        def body(g, carry):
            r0 = pl.multiple_of((wid * per_w + g * G) * IDXW, IDXW)
            for j in range(G):
                pltpu.sync_copy(
                    idx_hbm.at[pl.ds(pl.multiple_of(r0 + j * IDXW, IDXW), IDXW)],
                    idx_vs[j])
            copies = [
                pltpu.async_copy(tbl_hbm.at[idx_vs[j]], rows_vs[j], sem)
                for j in range(G)
            ]
            for c in copies:
                c.wait()
            for j in range(G):
                pltpu.sync_copy(
                    rows_vs[j],
                    out_hbm.at[pl.ds(pl.multiple_of(r0 + j * IDXW, IDXW), IDXW)])
            return carry

        lax.fori_loop(0, CHUNKS, body, 0)

    return gather_k


def _combine(sup_flat, par, num_values, smat, bias, B):
    BB = 512
    D = CAT_OUT + NUM_OUT

    def body(sup_ref, par_ref, nv_ref, s_ref, b_ref, out_ref):
        p = par_ref[...]
        parts = []
        for g in range(N_CAT):
            lo = sup_ref[:, g * LANES:g * LANES + CAT_DIM]
            hi = sup_ref[:, g * LANES + CAT_DIM:g * LANES + 2 * CAT_DIM]
            pg = p[:, g:g + 1]
            parts.append(jnp.where(pg > 0.5, hi, lo))
        num = jax.nn.relu(
            jnp.dot(nv_ref[...], s_ref[...],
                    preferred_element_type=jnp.float32,
                    precision=jax.lax.Precision.HIGHEST) + b_ref[...])
        out_ref[...] = jnp.concatenate(parts + [num], axis=1)

    return pl.pallas_call(
        body,
        grid=(B // BB,),
        in_specs=[
            pl.BlockSpec((BB, N_CAT * LANES), lambda i: (i, 0)),
            pl.BlockSpec((BB, N_CAT), lambda i: (i, 0)),
            pl.BlockSpec((BB, N_NUM), lambda i: (i, 0)),
            pl.BlockSpec((N_NUM, NUM_OUT), lambda i: (0, 0)),
            pl.BlockSpec((1, NUM_OUT), lambda i: (0, 0)),
        ],
        out_specs=pl.BlockSpec((BB, D), lambda i: (i, 0)),
        out_shape=jax.ShapeDtypeStruct((B, D), jnp.float32),
    )(sup_flat, par, num_values, smat, bias)


def kernel(cat_indices, num_values, cat_tables, num_w, num_b):
    B = cat_indices.shape[0]
    tbl_t = jnp.transpose(cat_tables, (0, 2, 1))
    packed = _pack_table(tbl_t)
    tbl2 = packed.reshape(N_CAT * SUP_PER_SLAB, LANES)
    offs2 = (jnp.arange(N_CAT, dtype=jnp.int32) * SUP_PER_SLAB)[None, :]
    hi_half = (cat_indices >= SUP_PER_SLAB).astype(jnp.int32)
    sidx = (cat_indices - hi_half * SUP_PER_SLAB + offs2).reshape(B * N_CAT)
    par = hi_half.astype(jnp.float32)
    sup = _make_gather(B)(tbl2, sidx)
    sup_flat = sup.reshape(B, N_CAT * LANES)
    j = jnp.arange(N_NUM)
    smat = jnp.zeros((N_NUM, NUM_OUT), jnp.float32).at[
        j[:, None], j[:, None] * NUM_DIM + jnp.arange(NUM_DIM)[None, :]
    ].set(num_w)
    bias = num_b.reshape(1, NUM_OUT)
    return _combine(sup_flat, par, num_values, smat, bias, B)


# speedup vs baseline: 4.4852x; 1.1846x over previous
import functools

import jax
import jax.numpy as jnp
from jax import lax
from jax.experimental import pallas as pl
from jax.experimental.pallas import tpu as pltpu
from jax.experimental.pallas import tpu_sc as plsc

N_CAT = 26
CARD1 = 100001
CAT_DIM = 36
N_NUM = 13
NUM_DIM = 8
CAT_OUT = N_CAT * CAT_DIM
NUM_OUT = N_NUM * NUM_DIM
IDXW = 128
LANES = 128

PACK_SUP = 2048
PACK_STEPS = 17
SUP_PER_SLAB = PACK_SUP * PACK_STEPS
PACKED_ROWS = N_CAT * SUP_PER_SLAB


def _pack_table(tbl_t):
    def body(a_ref, b_ref, c_ref, o_ref):
        a = a_ref[0]
        b = b_ref[0]
        c = c_ref[0]
        o_ref[...] = jnp.concatenate(
            [a.T, b.T, c.T, jnp.zeros((PACK_SUP, LANES - 3 * CAT_DIM),
                                      jnp.float32)], axis=1)

    return pl.pallas_call(
        body,
        grid=(N_CAT, PACK_STEPS),
        in_specs=[
            pl.BlockSpec((1, CAT_DIM, PACK_SUP), lambda i, j: (i, 0, j)),
            pl.BlockSpec((1, CAT_DIM, PACK_SUP),
                         lambda i, j: (i, 0, j + PACK_STEPS)),
            pl.BlockSpec((1, CAT_DIM, PACK_SUP),
                         lambda i, j: (i, 0,
                                       jnp.minimum(j + 2 * PACK_STEPS, 48))),
        ],
        out_specs=pl.BlockSpec((PACK_SUP, LANES),
                               lambda i, j: (i * PACK_STEPS + j, 0)),
        out_shape=jax.ShapeDtypeStruct((PACKED_ROWS, LANES), jnp.float32),
    )(tbl_t, tbl_t, tbl_t)


@functools.lru_cache(maxsize=None)
def _make_gather(B: int):
    info = plsc.get_sparse_core_info()
    NC, NS = info.num_cores, info.num_subcores
    NW = NC * NS
    TOTAL = B * N_CAT
    NROWS = TOTAL // IDXW
    per_w = NROWS // NW
    G = 4
    CHUNKS = per_w // G

    mesh = plsc.VectorSubcoreMesh(core_axis_name="c", subcore_axis_name="s")

    @functools.partial(
        pl.kernel,
        mesh=mesh,
        out_type=jax.ShapeDtypeStruct((TOTAL, LANES), jnp.float32),
        scratch_types=(
            [pltpu.VMEM((IDXW,), jnp.int32) for _ in range(G)]
            + [pltpu.VMEM((IDXW, LANES), jnp.float32) for _ in range(G)]
            + [pltpu.SemaphoreType.DMA]
        ),
    )
    def gather_k(tbl_hbm, idx_hbm, out_hbm, *scratch):
        idx_vs = scratch[:G]
        rows_vs = scratch[G:2 * G]
        sem = scratch[2 * G]
        wid = lax.axis_index("s") * NC + lax.axis_index("c")

        def body(g, carry):
            r0 = pl.multiple_of((wid * per_w + g * G) * IDXW, IDXW)
            for j in range(G):
                pltpu.sync_copy(
                    idx_hbm.at[pl.ds(pl.multiple_of(r0 + j * IDXW, IDXW), IDXW)],
                    idx_vs[j])
            copies = [
                pltpu.async_copy(tbl_hbm.at[idx_vs[j]], rows_vs[j], sem)
                for j in range(G)
            ]
            for c in copies:
                c.wait()
            for j in range(G):
                pltpu.sync_copy(
                    rows_vs[j],
                    out_hbm.at[pl.ds(pl.multiple_of(r0 + j * IDXW, IDXW), IDXW)])
            return carry

        lax.fori_loop(0, CHUNKS, body, 0)

    return gather_k


def _combine(sup_flat, par, num_values, smat, bias, B):
    BB = 512
    D = CAT_OUT + NUM_OUT

    def body(sup_ref, par_ref, nv_ref, s_ref, b_ref, out_ref):
        p = par_ref[...]
        parts = []
        for g in range(N_CAT):
            x0 = sup_ref[:, g * LANES:g * LANES + CAT_DIM]
            x1 = sup_ref[:, g * LANES + CAT_DIM:g * LANES + 2 * CAT_DIM]
            x2 = sup_ref[:, g * LANES + 2 * CAT_DIM:g * LANES + 3 * CAT_DIM]
            pg = p[:, g:g + 1]
            parts.append(jnp.where(pg < 0.5, x0, jnp.where(pg < 1.5, x1, x2)))
        num = jax.nn.relu(
            jnp.dot(nv_ref[...], s_ref[...],
                    preferred_element_type=jnp.float32,
                    precision=jax.lax.Precision.HIGHEST) + b_ref[...])
        out_ref[...] = jnp.concatenate(parts + [num], axis=1)

    return pl.pallas_call(
        body,
        grid=(B // BB,),
        in_specs=[
            pl.BlockSpec((BB, N_CAT * LANES), lambda i: (i, 0)),
            pl.BlockSpec((BB, N_CAT), lambda i: (i, 0)),
            pl.BlockSpec((BB, N_NUM), lambda i: (i, 0)),
            pl.BlockSpec((N_NUM, NUM_OUT), lambda i: (0, 0)),
            pl.BlockSpec((1, NUM_OUT), lambda i: (0, 0)),
        ],
        out_specs=pl.BlockSpec((BB, D), lambda i: (i, 0)),
        out_shape=jax.ShapeDtypeStruct((B, D), jnp.float32),
    )(sup_flat, par, num_values, smat, bias)


def kernel(cat_indices, num_values, cat_tables, num_w, num_b):
    B = cat_indices.shape[0]
    tbl_t = jnp.transpose(cat_tables, (0, 2, 1))
    tbl2 = _pack_table(tbl_t)
    offs2 = (jnp.arange(N_CAT, dtype=jnp.int32) * SUP_PER_SLAB)[None, :]
    third = cat_indices // SUP_PER_SLAB
    sidx = (cat_indices - third * SUP_PER_SLAB + offs2).reshape(B * N_CAT)
    par = third.astype(jnp.float32)
    sup = _make_gather(B)(tbl2, sidx)
    sup_flat = sup.reshape(B, N_CAT * LANES)
    j = jnp.arange(N_NUM)
    smat = jnp.zeros((N_NUM, NUM_OUT), jnp.float32).at[
        j[:, None], j[:, None] * NUM_DIM + jnp.arange(NUM_DIM)[None, :]
    ].set(num_w)
    bias = num_b.reshape(1, NUM_OUT)
    return _combine(sup_flat, par, num_values, smat, bias, B)


# speedup vs baseline: 5.0896x; 1.1348x over previous
import functools

import jax
import jax.numpy as jnp
from jax import lax
from jax.experimental import pallas as pl
from jax.experimental.pallas import tpu as pltpu
from jax.experimental.pallas import tpu_sc as plsc

N_CAT = 26
CARD1 = 100001
CAT_DIM = 36
N_NUM = 13
NUM_DIM = 8
CAT_OUT = N_CAT * CAT_DIM
NUM_OUT = N_NUM * NUM_DIM
IDXW = 128
LANES = 128

PACK_SUP = 2048
PACK_STEPS = 17
SUP_PER_SLAB = PACK_SUP * PACK_STEPS
PACKED_ROWS = N_CAT * SUP_PER_SLAB


def _pack_table(tbl_t):
    def body(a_ref, b_ref, c_ref, o_ref):
        a = a_ref[0]
        b = b_ref[0]
        c = c_ref[0]
        o_ref[...] = jnp.concatenate(
            [a.T, b.T, c.T, jnp.zeros((PACK_SUP, LANES - 3 * CAT_DIM),
                                      jnp.float32)], axis=1)

    return pl.pallas_call(
        body,
        grid=(N_CAT, PACK_STEPS),
        in_specs=[
            pl.BlockSpec((1, CAT_DIM, PACK_SUP), lambda i, j: (i, 0, j)),
            pl.BlockSpec((1, CAT_DIM, PACK_SUP),
                         lambda i, j: (i, 0, j + PACK_STEPS)),
            pl.BlockSpec((1, CAT_DIM, PACK_SUP),
                         lambda i, j: (i, 0,
                                       jnp.minimum(j + 2 * PACK_STEPS, 48))),
        ],
        out_specs=pl.BlockSpec((PACK_SUP, LANES),
                               lambda i, j: (i * PACK_STEPS + j, 0)),
        out_shape=jax.ShapeDtypeStruct((PACKED_ROWS, LANES), jnp.float32),
    )(tbl_t, tbl_t, tbl_t)


@functools.lru_cache(maxsize=None)
def _make_gather(B: int):
    info = plsc.get_sparse_core_info()
    NC, NS = info.num_cores, info.num_subcores
    NW = NC * NS
    TOTAL = B * N_CAT
    NROWS = TOTAL // IDXW
    per_w = NROWS // NW
    G = 4
    CHUNKS = per_w // G

    mesh = plsc.VectorSubcoreMesh(core_axis_name="c", subcore_axis_name="s")

    @functools.partial(
        pl.kernel,
        mesh=mesh,
        out_type=jax.ShapeDtypeStruct((TOTAL, LANES), jnp.float32),
        scratch_types=(
            [pltpu.VMEM((IDXW,), jnp.int32) for _ in range(G)]
            + [pltpu.VMEM((IDXW, LANES), jnp.float32) for _ in range(G)]
            + [pltpu.SemaphoreType.DMA]
        ),
    )
    def gather_k(tbl_hbm, idx_hbm, out_hbm, *scratch):
        idx_vs = scratch[:G]
        rows_vs = scratch[G:2 * G]
        sem = scratch[2 * G]
        wid = lax.axis_index("s") * NC + lax.axis_index("c")

        def body(g, carry):
            r0 = pl.multiple_of((wid * per_w + g * G) * IDXW, IDXW)
            for j in range(G):
                pltpu.sync_copy(
                    idx_hbm.at[pl.ds(pl.multiple_of(r0 + j * IDXW, IDXW), IDXW)],
                    idx_vs[j])
            copies = [
                pltpu.async_copy(tbl_hbm.at[idx_vs[j]], rows_vs[j], sem)
                for j in range(G)
            ]
            for c in copies:
                c.wait()
            for j in range(G):
                pltpu.sync_copy(
                    rows_vs[j],
                    out_hbm.at[pl.ds(pl.multiple_of(r0 + j * IDXW, IDXW), IDXW)])
            return carry

        lax.fori_loop(0, CHUNKS, body, 0)

    return gather_k


def _combine(sup_flat, par, num_values, smat, bias, B):
    BB = 512
    D = CAT_OUT + NUM_OUT

    def body(sup_ref, par_ref, nv_ref, s_ref, b_ref, out_ref):
        p = par_ref[...]
        parts = []
        for g in range(N_CAT):
            y = sup_ref[:, pl.ds(g * 8, 8), :][...]
            y = y.reshape(BB, LANES)
            x0 = y[:, 0:CAT_DIM]
            x1 = y[:, CAT_DIM:2 * CAT_DIM]
            x2 = y[:, 2 * CAT_DIM:3 * CAT_DIM]
            pg = p[:, g:g + 1]
            parts.append(jnp.where(pg < 0.5, x0, jnp.where(pg < 1.5, x1, x2)))
        num = jax.nn.relu(
            jnp.dot(nv_ref[...], s_ref[...],
                    preferred_element_type=jnp.float32,
                    precision=jax.lax.Precision.HIGHEST) + b_ref[...])
        out_ref[...] = jnp.concatenate(parts + [num], axis=1)

    return pl.pallas_call(
        body,
        grid=(B // BB,),
        in_specs=[
            pl.BlockSpec((BB // 8, N_CAT * 8, LANES), lambda i: (i, 0, 0)),
            pl.BlockSpec((BB, N_CAT), lambda i: (i, 0)),
            pl.BlockSpec((BB, N_NUM), lambda i: (i, 0)),
            pl.BlockSpec((N_NUM, NUM_OUT), lambda i: (0, 0)),
            pl.BlockSpec((1, NUM_OUT), lambda i: (0, 0)),
        ],
        out_specs=pl.BlockSpec((BB, D), lambda i: (i, 0)),
        out_shape=jax.ShapeDtypeStruct((B, D), jnp.float32),
    )(sup_flat, par, num_values, smat, bias)


def kernel(cat_indices, num_values, cat_tables, num_w, num_b):
    B = cat_indices.shape[0]
    tbl_t = jnp.transpose(cat_tables, (0, 2, 1))
    tbl2 = _pack_table(tbl_t)
    offs2 = (jnp.arange(N_CAT, dtype=jnp.int32) * SUP_PER_SLAB)[None, :]
    third = cat_indices // SUP_PER_SLAB
    sidx2 = cat_indices - third * SUP_PER_SLAB + offs2
    sidx = (sidx2.reshape(B // 8, 8, N_CAT)
            .transpose(0, 2, 1).reshape(B * N_CAT))
    par = third.astype(jnp.float32)
    sup = _make_gather(B)(tbl2, sidx)
    sup_flat = sup.reshape(B // 8, N_CAT * 8, LANES)
    j = jnp.arange(N_NUM)
    smat = jnp.zeros((N_NUM, NUM_OUT), jnp.float32).at[
        j[:, None], j[:, None] * NUM_DIM + jnp.arange(NUM_DIM)[None, :]
    ].set(num_w)
    bias = num_b.reshape(1, NUM_OUT)
    return _combine(sup_flat, par, num_values, smat, bias, B)


# speedup vs baseline: 5.2042x; 1.0225x over previous
import functools

import jax
import jax.numpy as jnp
from jax import lax
from jax.experimental import pallas as pl
from jax.experimental.pallas import tpu as pltpu
from jax.experimental.pallas import tpu_sc as plsc

N_CAT = 26
CARD1 = 100001
CAT_DIM = 36
N_NUM = 13
NUM_DIM = 8
CAT_OUT = N_CAT * CAT_DIM
NUM_OUT = N_NUM * NUM_DIM
IDXW = 128
LANES = 128

PACK_SUP = 4096
PACK_STEPS = 9
SUP_PER_SLAB = PACK_SUP * PACK_STEPS
PACKED_ROWS = N_CAT * SUP_PER_SLAB


def _pack_table(tbl_t):
    def body(a_ref, b_ref, c_ref, o_ref):
        a = a_ref[0]
        b = b_ref[0]
        c = c_ref[0]
        o_ref[...] = jnp.concatenate(
            [a.T, b.T, c.T, jnp.zeros((PACK_SUP, LANES - 3 * CAT_DIM),
                                      jnp.float32)], axis=1)

    return pl.pallas_call(
        body,
        grid=(N_CAT, PACK_STEPS),
        in_specs=[
            pl.BlockSpec((1, CAT_DIM, PACK_SUP), lambda i, j: (i, 0, j)),
            pl.BlockSpec((1, CAT_DIM, PACK_SUP),
                         lambda i, j: (i, 0, j + PACK_STEPS)),
            pl.BlockSpec((1, CAT_DIM, PACK_SUP),
                         lambda i, j: (i, 0,
                                       jnp.minimum(j + 2 * PACK_STEPS, 24))),
        ],
        out_specs=pl.BlockSpec((PACK_SUP, LANES),
                               lambda i, j: (i * PACK_STEPS + j, 0)),
        out_shape=jax.ShapeDtypeStruct((PACKED_ROWS, LANES), jnp.float32),
    )(tbl_t, tbl_t, tbl_t)


@functools.lru_cache(maxsize=None)
def _make_gather(B: int):
    info = plsc.get_sparse_core_info()
    NC, NS = info.num_cores, info.num_subcores
    NW = NC * NS
    TOTAL = B * N_CAT
    NROWS = TOTAL // IDXW
    per_w = NROWS // NW
    G = 4
    CHUNKS = per_w // G

    mesh = plsc.VectorSubcoreMesh(core_axis_name="c", subcore_axis_name="s")

    @functools.partial(
        pl.kernel,
        mesh=mesh,
        out_type=jax.ShapeDtypeStruct((TOTAL, LANES), jnp.float32),
        scratch_types=(
            [pltpu.VMEM((IDXW,), jnp.int32) for _ in range(G)]
            + [pltpu.VMEM((IDXW, LANES), jnp.float32) for _ in range(G)]
            + [pltpu.SemaphoreType.DMA]
        ),
    )
    def gather_k(tbl_hbm, idx_hbm, out_hbm, *scratch):
        idx_vs = scratch[:G]
        rows_vs = scratch[G:2 * G]
        sem = scratch[2 * G]
        wid = lax.axis_index("s") * NC + lax.axis_index("c")

        def body(g, carry):
            r0 = pl.multiple_of((wid * per_w + g * G) * IDXW, IDXW)
            for j in range(G):
                pltpu.sync_copy(
                    idx_hbm.at[pl.ds(pl.multiple_of(r0 + j * IDXW, IDXW), IDXW)],
                    idx_vs[j])
            copies = [
                pltpu.async_copy(tbl_hbm.at[idx_vs[j]], rows_vs[j], sem)
                for j in range(G)
            ]
            for c in copies:
                c.wait()
            for j in range(G):
                pltpu.sync_copy(
                    rows_vs[j],
                    out_hbm.at[pl.ds(pl.multiple_of(r0 + j * IDXW, IDXW), IDXW)])
            return carry

        lax.fori_loop(0, CHUNKS, body, 0)

    return gather_k


def _combine(sup_flat, par, num_values, smat, bias, B):
    BB = 512
    D = CAT_OUT + NUM_OUT

    def body(sup_ref, par_ref, nv_ref, s_ref, b_ref, out_ref):
        p = par_ref[...]
        parts = []
        for g in range(N_CAT):
            y = sup_ref[:, pl.ds(g * 8, 8), :][...]
            y = y.reshape(BB, LANES)
            x0 = y[:, 0:CAT_DIM]
            x1 = y[:, CAT_DIM:2 * CAT_DIM]
            x2 = y[:, 2 * CAT_DIM:3 * CAT_DIM]
            pg = p[:, g:g + 1]
            parts.append(jnp.where(pg < 0.5, x0, jnp.where(pg < 1.5, x1, x2)))
        num = jax.nn.relu(
            jnp.dot(nv_ref[...], s_ref[...],
                    preferred_element_type=jnp.float32,
                    precision=jax.lax.Precision.HIGHEST) + b_ref[...])
        out_ref[...] = jnp.concatenate(parts + [num], axis=1)

    return pl.pallas_call(
        body,
        grid=(B // BB,),
        in_specs=[
            pl.BlockSpec((BB // 8, N_CAT * 8, LANES), lambda i: (i, 0, 0)),
            pl.BlockSpec((BB, N_CAT), lambda i: (i, 0)),
            pl.BlockSpec((BB, N_NUM), lambda i: (i, 0)),
            pl.BlockSpec((N_NUM, NUM_OUT), lambda i: (0, 0)),
            pl.BlockSpec((1, NUM_OUT), lambda i: (0, 0)),
        ],
        out_specs=pl.BlockSpec((BB, D), lambda i: (i, 0)),
        out_shape=jax.ShapeDtypeStruct((B, D), jnp.float32),
    )(sup_flat, par, num_values, smat, bias)


def kernel(cat_indices, num_values, cat_tables, num_w, num_b):
    B = cat_indices.shape[0]
    tbl_t = jnp.transpose(cat_tables, (0, 2, 1))
    tbl2 = _pack_table(tbl_t)
    offs2 = (jnp.arange(N_CAT, dtype=jnp.int32) * SUP_PER_SLAB)[None, :]
    third = cat_indices // SUP_PER_SLAB
    sidx2 = cat_indices - third * SUP_PER_SLAB + offs2
    sidx = (sidx2.reshape(B // 8, 8, N_CAT)
            .transpose(0, 2, 1).reshape(B * N_CAT))
    par = third.astype(jnp.float32)
    sup = _make_gather(B)(tbl2, sidx)
    sup_flat = sup.reshape(B // 8, N_CAT * 8, LANES)
    j = jnp.arange(N_NUM)
    smat = jnp.zeros((N_NUM, NUM_OUT), jnp.float32).at[
        j[:, None], j[:, None] * NUM_DIM + jnp.arange(NUM_DIM)[None, :]
    ].set(num_w)
    bias = num_b.reshape(1, NUM_OUT)
    return _combine(sup_flat, par, num_values, smat, bias, B)
